# baseline (speedup 1.0000x reference)
import functools
import numpy as np
import jax
import jax.numpy as jnp
from jax import lax
from jax.experimental import pallas as pl
from jax.experimental.pallas import tpu as pltpu
from jax.experimental.pallas import tpu_sc as plsc

_DIM = 256
_G = 16
_DPG = _DIM // _G
_PAD = _DPG + 1

_NC = 2
_NS = 16
_NW = _NC * _NS

_ROWS = 32 * 1024
_ROWS_PER_W = _ROWS // _NW
_T = 128
_CHUNKS = _ROWS_PER_W // _T
_NBUF = 3


def _sc_body(x_hbm, idx_hbm, xq, idxb, in_sems, outi_sems):
    cid = lax.axis_index("c")
    sid = lax.axis_index("s")
    wid = sid * _NC + cid
    g_iota = lax.iota(jnp.int32, _G)
    col_base = g_iota * _DPG
    row0 = wid * _ROWS_PER_W

    def start_in(c, b):
        pltpu.async_copy(x_hbm.at[pl.ds(row0 + c * _T, _T), :],
                         xq.at[b], in_sems[b])

    def wait_in(c, b):
        pltpu.make_async_copy(x_hbm.at[pl.ds(row0 + c * _T, _T), :],
                              xq.at[b], in_sems[b]).wait()

    def start_out(c, b):
        pltpu.async_copy(idxb.at[b],
                         idx_hbm.at[:, pl.ds(row0 + c * _T, _T)],
                         outi_sems[b])

    def wait_out(c, b):
        pltpu.make_async_copy(idxb.at[b],
                              idx_hbm.at[:, pl.ds(row0 + c * _T, _T)],
                              outi_sems[b]).wait()

    def compute(b):
        xbuf = xq.at[b]
        idxbuf = idxb.at[b]

        @plsc.parallel_loop(0, _T, unroll=4)
        def tok_body(t):
            tv = jnp.full((_G,), t, jnp.int32)
            terms = []
            for j in range(_DPG):
                pv = (g_iota + j) & (_DPG - 1)
                cv = col_base + pv
                v = plsc.load_gather(xbuf, [tv, cv])
                m = v > 0
                wv = jnp.int32(1 << (_DPG - 1)) >> pv
                terms.append(jnp.where(m, wv, jnp.int32(0)))
            while len(terms) > 1:
                terms = [terms[k] + terms[k + 1]
                         for k in range(0, len(terms), 2)]
            plsc.store_scatter(idxbuf, [g_iota, tv], terms[0])

    start_in(0, 0)
    start_in(1, 1)
    for c in range(_CHUNKS):
        b = c % _NBUF
        wait_in(c, b)
        compute(b)
        start_out(c, b)
        n = c + 2
        if n < _CHUNKS:
            bn = n % _NBUF
            if n >= _NBUF:
                wait_out(n - _NBUF, bn)
            start_in(n, bn)
    for c in range(max(0, _CHUNKS - _NBUF), _CHUNKS):
        wait_out(c, c % _NBUF)


def _sc_indices(xg):
    mesh = plsc.VectorSubcoreMesh(core_axis_name="c", subcore_axis_name="s")
    run = pl.kernel(
        _sc_body,
        out_type=jax.ShapeDtypeStruct((_G, _ROWS), jnp.int32),
        mesh=mesh,
        scratch_types=[
            pltpu.VMEM((_NBUF, _T, _DIM), jnp.float32),
            pltpu.VMEM((_NBUF, _G, _T), jnp.int32),
            [pltpu.SemaphoreType.DMA] * _NBUF,
            [pltpu.SemaphoreType.DMA] * _NBUF,
        ],
        compiler_params=pltpu.CompilerParams(needs_layout_passes=False,
                                             disable_bounds_checks=True),
    )
    return run(xg)


_ROWS_PER_TC_BLOCK = 2048


def _tc_body(x_ref, q_ref):
    x = x_ref[...]
    q_ref[...] = jnp.where(x > 0, jnp.float32(0.25), jnp.float32(-0.25))


def _tc_quantize(x3d):
    b, n, dim = x3d.shape
    bb = 8
    return pl.pallas_call(
        _tc_body,
        grid=(b // bb,),
        in_specs=[pl.BlockSpec((bb, n, dim), lambda i: (i, 0, 0))],
        out_specs=pl.BlockSpec((bb, n, dim), lambda i: (i, 0, 0)),
        out_shape=jax.ShapeDtypeStruct((b, n, dim), jnp.float32),
    )(x3d)


def kernel(x):
    b, n, dim = x.shape
    xg = x.reshape(-1, _DIM)
    idx = _sc_indices(xg)
    quantized = _tc_quantize(x)
    all_indices = idx.reshape(_G, b, n)
    aux_losses = jnp.zeros((_G,), dtype=jnp.float32)
    return (quantized, all_indices, aux_losses)


# speedup vs baseline: 1.0448x; 1.0448x over previous
import functools
import numpy as np
import jax
import jax.numpy as jnp
from jax import lax
from jax.experimental import pallas as pl
from jax.experimental.pallas import tpu as pltpu
from jax.experimental.pallas import tpu_sc as plsc

_DIM = 256
_G = 16
_DPG = _DIM // _G
_PAD = _DPG + 1

_NC = 2
_NS = 16
_NW = _NC * _NS

_ROWS = 32 * 1024
_ROWS_PER_W = _ROWS // _NW
_T = 128
_CHUNKS = _ROWS_PER_W // _T
_NBUF = 3


def _sc_body(x_hbm, idx_hbm, xq, idxb, in_sems, outi_sems):
    cid = lax.axis_index("c")
    sid = lax.axis_index("s")
    wid = sid * _NC + cid
    g_iota = lax.iota(jnp.int32, _G)
    col_base = g_iota * _DPG
    row0 = wid * _ROWS_PER_W

    def start_in(c, b):
        pltpu.async_copy(x_hbm.at[pl.ds(row0 + c * _T, _T), :],
                         xq.at[b], in_sems[b])

    def wait_in(c, b):
        pltpu.make_async_copy(x_hbm.at[pl.ds(row0 + c * _T, _T), :],
                              xq.at[b], in_sems[b]).wait()

    def start_out(c, b):
        pltpu.async_copy(idxb.at[b, :, pl.ds(0, _T)],
                         idx_hbm.at[:, pl.ds(row0 + c * _T, _T)],
                         outi_sems[b])

    def wait_out(c, b):
        pltpu.make_async_copy(idxb.at[b, :, pl.ds(0, _T)],
                              idx_hbm.at[:, pl.ds(row0 + c * _T, _T)],
                              outi_sems[b]).wait()

    def compute(b):
        xbuf = xq.at[b]
        idxbuf = idxb.at[b]

        @plsc.parallel_loop(0, _T, unroll=4)
        def tok_body(t):
            tv = jnp.full((_G,), t, jnp.int32)
            terms = []
            for j in range(_DPG):
                pv = (g_iota + j) & (_DPG - 1)
                cv = col_base + pv
                v = plsc.load_gather(xbuf, [tv, cv])
                m = v > 0
                wv = jnp.int32(1 << (_DPG - 1)) >> pv
                terms.append(jnp.where(m, wv, jnp.int32(0)))
            while len(terms) > 1:
                terms = [terms[k] + terms[k + 1]
                         for k in range(0, len(terms), 2)]
            plsc.store_scatter(idxbuf, [g_iota, tv], terms[0])

    start_in(0, 0)
    start_in(1, 1)
    for c in range(_CHUNKS):
        b = c % _NBUF
        wait_in(c, b)
        compute(b)
        start_out(c, b)
        n = c + 2
        if n < _CHUNKS:
            bn = n % _NBUF
            if n >= _NBUF:
                wait_out(n - _NBUF, bn)
            start_in(n, bn)
    for c in range(max(0, _CHUNKS - _NBUF), _CHUNKS):
        wait_out(c, c % _NBUF)


def _sc_indices(xg):
    mesh = plsc.VectorSubcoreMesh(core_axis_name="c", subcore_axis_name="s")
    run = pl.kernel(
        _sc_body,
        out_type=jax.ShapeDtypeStruct((_G, _ROWS), jnp.int32),
        mesh=mesh,
        scratch_types=[
            pltpu.VMEM((_NBUF, _T, _DIM), jnp.float32),
            pltpu.VMEM((_NBUF, _G, _T + 1), jnp.int32),
            [pltpu.SemaphoreType.DMA] * _NBUF,
            [pltpu.SemaphoreType.DMA] * _NBUF,
        ],
        compiler_params=pltpu.CompilerParams(needs_layout_passes=False,
                                             disable_bounds_checks=True),
    )
    return run(xg)


_ROWS_PER_TC_BLOCK = 2048


def _tc_body(x_ref, q_ref):
    x = x_ref[...]
    q_ref[...] = jnp.where(x > 0, jnp.float32(0.25), jnp.float32(-0.25))


def _tc_quantize(xg):
    rpb = _ROWS_PER_TC_BLOCK
    return pl.pallas_call(
        _tc_body,
        grid=(_ROWS // rpb,),
        in_specs=[pl.BlockSpec((rpb, _DIM), lambda i: (i, 0))],
        out_specs=pl.BlockSpec((rpb, _DIM), lambda i: (i, 0)),
        out_shape=jax.ShapeDtypeStruct((_ROWS, _DIM), jnp.float32),
    )(xg)


def kernel(x):
    b, n, dim = x.shape
    xg = x.reshape(-1, _DIM)
    idx = _sc_indices(xg)
    qf = _tc_quantize(xg)
    quantized = qf.reshape(b, n, dim)
    all_indices = idx.reshape(_G, b, n)
    aux_losses = jnp.zeros((_G,), dtype=jnp.float32)
    return (quantized, all_indices, aux_losses)


# speedup vs baseline: 1.0460x; 1.0011x over previous
import functools
import numpy as np
import jax
import jax.numpy as jnp
from jax import lax
from jax.experimental import pallas as pl
from jax.experimental.pallas import tpu as pltpu
from jax.experimental.pallas import tpu_sc as plsc

_DIM = 256
_G = 16
_DPG = _DIM // _G
_PAD = _DPG + 1

_NC = 2
_NS = 16
_NW = _NC * _NS

_ROWS = 32 * 1024
_ROWS_PER_W = _ROWS // _NW
_T = 128
_CHUNKS = _ROWS_PER_W // _T
_NBUF = 3


def _sc_body(x_hbm, idx_hbm, xq, idxb, in_sems, outi_sems):
    cid = lax.axis_index("c")
    sid = lax.axis_index("s")
    wid = sid * _NC + cid
    g_iota = lax.iota(jnp.int32, _G)
    col_base = g_iota * _DPG
    row0 = wid * _ROWS_PER_W

    def start_in(c, b):
        pltpu.async_copy(x_hbm.at[pl.ds(row0 + c * _T, _T), :],
                         xq.at[b], in_sems[b])

    def wait_in(c, b):
        pltpu.make_async_copy(x_hbm.at[pl.ds(row0 + c * _T, _T), :],
                              xq.at[b], in_sems[b]).wait()

    def start_out(c, b):
        pltpu.async_copy(idxb.at[b, :, pl.ds(0, _T)],
                         idx_hbm.at[:, pl.ds(row0 + c * _T, _T)],
                         outi_sems[b])

    def wait_out(c, b):
        pltpu.make_async_copy(idxb.at[b, :, pl.ds(0, _T)],
                              idx_hbm.at[:, pl.ds(row0 + c * _T, _T)],
                              outi_sems[b]).wait()

    def compute(b):
        xbuf = xq.at[b]
        idxbuf = idxb.at[b]

        @plsc.parallel_loop(0, _T, unroll=4)
        def tok_body(t):
            tv = jnp.full((_G,), t, jnp.int32)
            terms = []
            for j in range(_DPG):
                cv = col_base + ((g_iota + j) & (_DPG - 1))
                v = plsc.load_gather(xbuf, [tv, cv])
                m = v > 0
                wv = jnp.int32(1 << (_DPG - 1)) >> (cv & (_DPG - 1))
                terms.append(jnp.where(m, wv, jnp.int32(0)))
            while len(terms) > 1:
                terms = [terms[k] + terms[k + 1]
                         for k in range(0, len(terms), 2)]
            plsc.store_scatter(idxbuf, [g_iota, tv], terms[0])

    start_in(0, 0)
    start_in(1, 1)
    for c in range(_CHUNKS):
        b = c % _NBUF
        wait_in(c, b)
        compute(b)
        start_out(c, b)
        n = c + 2
        if n < _CHUNKS:
            bn = n % _NBUF
            if n >= _NBUF:
                wait_out(n - _NBUF, bn)
            start_in(n, bn)
    for c in range(max(0, _CHUNKS - _NBUF), _CHUNKS):
        wait_out(c, c % _NBUF)


def _sc_indices(xg):
    mesh = plsc.VectorSubcoreMesh(core_axis_name="c", subcore_axis_name="s")
    run = pl.kernel(
        _sc_body,
        out_type=jax.ShapeDtypeStruct((_G, _ROWS), jnp.int32),
        mesh=mesh,
        scratch_types=[
            pltpu.VMEM((_NBUF, _T, _DIM), jnp.float32),
            pltpu.VMEM((_NBUF, _G, _T + 1), jnp.int32),
            [pltpu.SemaphoreType.DMA] * _NBUF,
            [pltpu.SemaphoreType.DMA] * _NBUF,
        ],
        compiler_params=pltpu.CompilerParams(needs_layout_passes=False,
                                             disable_bounds_checks=True),
    )
    return run(xg)


_ROWS_PER_TC_BLOCK = 2048


def _tc_body(x_ref, q_ref):
    x = x_ref[...]
    q_ref[...] = jnp.where(x > 0, jnp.float32(0.25), jnp.float32(-0.25))


def _tc_quantize(xg):
    rpb = _ROWS_PER_TC_BLOCK
    return pl.pallas_call(
        _tc_body,
        grid=(_ROWS // rpb,),
        in_specs=[pl.BlockSpec((rpb, _DIM), lambda i: (i, 0))],
        out_specs=pl.BlockSpec((rpb, _DIM), lambda i: (i, 0)),
        out_shape=jax.ShapeDtypeStruct((_ROWS, _DIM), jnp.float32),
    )(xg)


def kernel(x):
    b, n, dim = x.shape
    xg = x.reshape(-1, _DIM)
    idx = _sc_indices(xg)
    qf = _tc_quantize(xg)
    quantized = qf.reshape(b, n, dim)
    all_indices = idx.reshape(_G, b, n)
    aux_losses = jnp.zeros((_G,), dtype=jnp.float32)
    return (quantized, all_indices, aux_losses)


# speedup vs baseline: 1.0492x; 1.0031x over previous
import jax
import jax.numpy as jnp
from jax import lax
from jax.experimental import pallas as pl
from jax.experimental.pallas import tpu as pltpu
from jax.experimental.pallas import tpu_sc as plsc

_DIM = 256
_G = 16
_DPG = _DIM // _G

_NC = 2
_NS = 16
_NW = _NC * _NS

_ROWS = 32 * 1024
_ROWS_PER_W = _ROWS // _NW
_T = 128
_CHUNKS = _ROWS_PER_W // _T
_NBUF = 3


def _sc_body(x_hbm, idx_hbm, xq, idxb, in_sems, outi_sems):
    cid = lax.axis_index("c")
    sid = lax.axis_index("s")
    wid = sid * _NC + cid
    g_iota = lax.iota(jnp.int32, _G)
    col_base = g_iota * _DPG
    row0 = wid * _ROWS_PER_W

    def start_in(c, b):
        pltpu.async_copy(x_hbm.at[pl.ds(row0 + c * _T, _T), :],
                         xq.at[b], in_sems[b])

    def wait_in(c, b):
        pltpu.make_async_copy(x_hbm.at[pl.ds(row0 + c * _T, _T), :],
                              xq.at[b], in_sems[b]).wait()

    def start_out(c, b):
        pltpu.async_copy(idxb.at[b, :, pl.ds(0, _T)],
                         idx_hbm.at[:, pl.ds(row0 + c * _T, _T)],
                         outi_sems[b])

    def wait_out(c, b):
        pltpu.make_async_copy(idxb.at[b, :, pl.ds(0, _T)],
                              idx_hbm.at[:, pl.ds(row0 + c * _T, _T)],
                              outi_sems[b]).wait()

    def compute(b):
        xbuf = xq.at[b]
        idxbuf = idxb.at[b]

        @plsc.parallel_loop(0, _T, unroll=4)
        def tok_body(t):
            tv = jnp.full((_G,), t, jnp.int32)
            terms = []
            for j in range(_DPG):
                cv = col_base + ((g_iota + j) & (_DPG - 1))
                v = plsc.load_gather(xbuf, [tv, cv])
                m = v > 0
                wv = jnp.int32(1 << (_DPG - 1)) >> (cv & (_DPG - 1))
                terms.append(jnp.where(m, wv, jnp.int32(0)))
            while len(terms) > 1:
                terms = [terms[k] + terms[k + 1]
                         for k in range(0, len(terms), 2)]
            plsc.store_scatter(idxbuf, [g_iota, tv], terms[0])

    start_in(0, 0)
    start_in(1, 1)
    for c in range(_CHUNKS):
        b = c % _NBUF
        wait_in(c, b)
        compute(b)
        start_out(c, b)
        n = c + 2
        if n < _CHUNKS:
            bn = n % _NBUF
            if n >= _NBUF:
                wait_out(n - _NBUF, bn)
            start_in(n, bn)
    for c in range(max(0, _CHUNKS - _NBUF), _CHUNKS):
        wait_out(c, c % _NBUF)


def _sc_indices(xg):
    mesh = plsc.VectorSubcoreMesh(core_axis_name="c", subcore_axis_name="s")
    run = pl.kernel(
        _sc_body,
        out_type=jax.ShapeDtypeStruct((_G, _ROWS), jnp.int32),
        mesh=mesh,
        scratch_types=[
            pltpu.VMEM((_NBUF, _T, _DIM), jnp.float32),
            pltpu.VMEM((_NBUF, _G, _T + 1), jnp.int32),
            [pltpu.SemaphoreType.DMA] * _NBUF,
            [pltpu.SemaphoreType.DMA] * _NBUF,
        ],
        compiler_params=pltpu.CompilerParams(needs_layout_passes=False,
                                             disable_bounds_checks=True),
    )
    return run(xg)


_ROWS_PER_TC_BLOCK = 2048


def _tc_body(x_ref, q_ref):
    x = x_ref[...]
    q_ref[...] = jnp.where(x > 0, jnp.float32(0.25), jnp.float32(-0.25))


def _tc_quantize(xg):
    rpb = _ROWS_PER_TC_BLOCK
    return pl.pallas_call(
        _tc_body,
        grid=(_ROWS // rpb,),
        in_specs=[pl.BlockSpec((rpb, _DIM), lambda i: (i, 0))],
        out_specs=pl.BlockSpec((rpb, _DIM), lambda i: (i, 0)),
        out_shape=jax.ShapeDtypeStruct((_ROWS, _DIM), jnp.float32),
    )(xg)


def kernel(x):
    b, n, dim = x.shape
    xg = x.reshape(-1, _DIM)
    idx = _sc_indices(xg)
    qf = _tc_quantize(xg)
    quantized = qf.reshape(b, n, dim)
    all_indices = idx.reshape(_G, b, n)
    aux_losses = jnp.zeros((_G,), dtype=jnp.float32)
    return (quantized, all_indices, aux_losses)
